# baseline (speedup 1.0000x reference)
import jax
import jax.numpy as jnp
import numpy as np
from jax import lax
from jax.experimental import pallas as pl
from jax.experimental.pallas import tpu as pltpu
from jax.experimental.pallas import tpu_sc as plsc

N = 5000
NPAD = 5120
NCHUNK = NPAD // 16
NEG = np.float32(-1e30)
NEG_HALF = np.float32(-5e29)
SCORE_T = np.float32(0.2)
IOU_T = np.float32(0.5)
HUMAN = np.int32(1)
MAXH = np.int32(15)
MAXO = np.int32(15)
NOUT = 30

_GDN = lax.GatherDimensionNumbers(
    offset_dims=(), collapsed_slice_dims=(0,), start_index_map=(0,))


def _perm(x, idx):
    return lax.gather(x, idx[:, None], _GDN, (1,),
                      mode=lax.GatherScatterMode.PROMISE_IN_BOUNDS)


def _argmax_splat(m, mi, lanes):
    for k in (8, 4, 2, 1):
        pidx = lanes ^ k
        pm = _perm(m, pidx)
        pmi = _perm(mi, pidx)
        better = (pm > m) | ((pm == m) & (pmi < mi))
        m = jnp.where(better, pm, m)
        mi = jnp.where(better, pmi, mi)
    return m, mi


def _sc_nms(x1h, y1h, x2h, y2h, sh, labh, pack_o, olab_o, oval_o,
            x1v, y1v, x2v, y2v, sv, labv,
            ox1v, oy1v, ox2v, oy2v, keyv, fillv,
            pack_v, olab_v, oval_v, msc, misc, fmisc):
    is0 = (lax.axis_index("c") == 0) & (lax.axis_index("s") == 0)

    @pl.when(is0)
    def _():
        lanes = lax.iota(jnp.int32, 16)

        pltpu.sync_copy(x1h, x1v)
        pltpu.sync_copy(y1h, y1v)
        pltpu.sync_copy(x2h, x2v)
        pltpu.sync_copy(y2h, y2v)
        pltpu.sync_copy(sh, sv)
        pltpu.sync_copy(labh, labv)

        def s1(c, m):
            d = pl.ds(c * 16, 16)
            return jnp.maximum(m, jnp.maximum(x2v[d], y2v[d]))
        mcv = lax.fori_loop(0, NCHUNK, s1, jnp.full((16,), NEG, jnp.float32))
        for k in (8, 4, 2, 1):
            mcv = jnp.maximum(mcv, _perm(mcv, lanes ^ k))
        mcv = mcv + 1.0

        def s2(c, carry):
            m, mi = carry
            d = pl.ds(c * 16, 16)
            idxv = c * 16 + lanes
            off = labv[d].astype(jnp.float32) * mcv
            a = x1v[d] + off
            b = y1v[d] + off
            cc = x2v[d] + off
            dd = y2v[d] + off
            ox1v[d] = a
            oy1v[d] = b
            ox2v[d] = cc
            oy2v[d] = dd
            sc = sv[d]
            vmask = sc >= SCORE_T
            keyc = jnp.where(vmask, sc, NEG)
            keyv[d] = keyc
            idxf = idxv.astype(jnp.float32)
            fillv[d] = jnp.where(vmask, sc,
                                 jnp.where(idxv < N, -(idxf + 2.0), NEG))
            upd = keyc > m
            return jnp.where(upd, keyc, m), jnp.where(upd, idxv, mi)

        m0, mi0 = lax.fori_loop(
            0, NCHUNK, s2,
            (jnp.full((16,), NEG, jnp.float32), jnp.zeros((16,), jnp.int32)))
        m0, mi0 = _argmax_splat(m0, mi0, lanes)
        msc[...] = m0
        misc[...] = mi0
        fmisc[...] = jnp.zeros((16,), jnp.int32)

        def body(t, carry):
            h, o = carry
            gm = msc[...][0]
            gi = misc[...][0]
            active = gm > NEG_HALF

            @pl.when(jnp.logical_not(active))
            def _():
                def fsweep(c, fcarry):
                    m, mi = fcarry
                    fc = fillv[pl.ds(c * 16, 16)]
                    upd = fc > m
                    return (jnp.where(upd, fc, m),
                            jnp.where(upd, c * 16 + lanes, mi))
                fm, fmi = lax.fori_loop(
                    0, NCHUNK, fsweep,
                    (jnp.full((16,), NEG, jnp.float32),
                     jnp.zeros((16,), jnp.int32)))
                fm, fmi = _argmax_splat(fm, fmi, lanes)
                fmisc[...] = fmi

            pick = jnp.where(active, gi, fmisc[...][0])

            fd = pl.ds(pick, 16)
            fold = fillv[fd]
            fillv[fd] = jnp.where(lanes == 0, NEG, fold)

            cx1 = x1v[fd][0]
            cy1 = y1v[fd][0]
            cx2 = x2v[fd][0]
            cy2 = y2v[fd][0]
            cs = sv[fd][0]
            plab = labv[fd][0]

            is_h = plab == HUMAN
            inc = active.astype(jnp.int32)
            h2 = h + jnp.where(is_h, inc, 0)
            o2 = o + jnp.where(is_h, 0, inc)


            vals = jnp.where(lanes == 0, cx1,
                   jnp.where(lanes == 1, cy1,
                   jnp.where(lanes == 2, cx2,
                   jnp.where(lanes == 3, cy2, cs))))
            pd = pl.ds(t * 5, 16)
            pold = pack_v[pd]
            pack_v[pd] = jnp.where(lanes < 5, vals, pold)
            od = pl.ds(t, 16)
            lold = olab_v[od]
            olab_v[od] = jnp.where(lanes == 0, plab, lold)
            vold = oval_v[od]
            oval_v[od] = jnp.where(lanes == 0, inc, vold)

            @pl.when(active)
            def _():
                pox1 = ox1v[fd][0]
                poy1 = oy1v[fd][0]
                pox2 = ox2v[fd][0]
                poy2 = oy2v[fd][0]
                poarea = (pox2 - pox1) * (poy2 - poy1)

                ph = 1 - jnp.minimum(jnp.abs(plab - HUMAN), 1)
                capchg = jnp.where(ph == 1, h2 == MAXH, o2 == MAXO)

                @pl.when(capchg)
                def _():
                    phv = jnp.full((16,), ph, jnp.int32)

                    def ksweep(c, _):
                        d = pl.ds(c * 16, 16)
                        labc = labv[d]
                        labh = 1 - jnp.minimum(jnp.abs(labc - HUMAN), 1)
                        keyv[d] = jnp.where(labh == phv, NEG, keyv[d])
                        return 0

                    lax.fori_loop(0, NCHUNK, ksweep, 0)

                def sweep(c, scarry):
                    m, mi = scarry
                    for k in range(8):
                        d = pl.ds(c * 128 + k * 16, 16)
                        idxv = c * 128 + k * 16 + lanes
                        a = ox1v[d]
                        b = oy1v[d]
                        cc = ox2v[d]
                        dd = oy2v[d]
                        ltx = jnp.maximum(pox1, a)
                        lty = jnp.maximum(poy1, b)
                        rbx = jnp.minimum(pox2, cc)
                        rby = jnp.minimum(poy2, dd)
                        w = jnp.maximum(rbx - ltx, 0.0)
                        hh = jnp.maximum(rby - lty, 0.0)
                        inter = w * hh
                        union = poarea + (cc - a) * (dd - b) - inter
                        iou = inter / jnp.maximum(union, np.float32(1e-8))
                        keyc = jnp.where(iou > IOU_T, NEG, keyv[d])
                        keyv[d] = keyc
                        upd = keyc > m
                        m = jnp.where(upd, keyc, m)
                        mi = jnp.where(upd, idxv, mi)
                    return m, mi

                m, mi = lax.fori_loop(
                    0, NCHUNK // 8, sweep,
                    (jnp.full((16,), NEG, jnp.float32),
                     jnp.zeros((16,), jnp.int32)))
                m, mi = _argmax_splat(m, mi, lanes)
                msc[...] = m
                misc[...] = mi

            return h2, o2

        lax.fori_loop(0, NOUT, body, (jnp.int32(0), jnp.int32(0)))

        pltpu.sync_copy(pack_v, pack_o)
        pltpu.sync_copy(olab_v, olab_o)
        pltpu.sync_copy(oval_v, oval_o)


@jax.jit
def kernel(boxes, scores, labels):
    pad = NPAD - N
    x1 = jnp.pad(boxes[:, 0], (0, pad))
    y1 = jnp.pad(boxes[:, 1], (0, pad))
    x2 = jnp.pad(boxes[:, 2], (0, pad))
    y2 = jnp.pad(boxes[:, 3], (0, pad))
    sp = jnp.pad(scores, (0, pad))
    lp = jnp.pad(labels, (0, pad))

    mesh = plsc.VectorSubcoreMesh(core_axis_name="c", subcore_axis_name="s")
    f = pl.kernel(
        _sc_nms, mesh=mesh,
        out_type=[
            jax.ShapeDtypeStruct((176,), jnp.float32),
            jax.ShapeDtypeStruct((48,), jnp.int32),
            jax.ShapeDtypeStruct((48,), jnp.int32),
        ],
        scratch_types=(
            [pltpu.VMEM((NPAD,), jnp.float32) for _ in range(5)]
            + [pltpu.VMEM((NPAD,), jnp.int32)]
            + [pltpu.VMEM((NPAD,), jnp.float32) for _ in range(6)]
            + [pltpu.VMEM((176,), jnp.float32),
               pltpu.VMEM((48,), jnp.int32),
               pltpu.VMEM((48,), jnp.int32),
               pltpu.VMEM((16,), jnp.float32),
               pltpu.VMEM((16,), jnp.int32),
               pltpu.VMEM((16,), jnp.int32)]),
    )
    pack, olab, oval = f(x1, y1, x2, y2, sp, lp)
    packed = pack[:150].reshape(30, 5)
    return packed, olab[:30], oval[:30].astype(bool)


# speedup vs baseline: 2.8241x; 2.8241x over previous
import jax
import jax.numpy as jnp
import numpy as np
from jax import lax
from jax.experimental import pallas as pl
from jax.experimental.pallas import tpu as pltpu
from jax.experimental.pallas import tpu_sc as plsc

N = 5000
NPAD = 5120
NCHUNK = NPAD // 16
NEG = np.float32(-1e30)
NEG_HALF = np.float32(-5e29)
SCORE_T = np.float32(0.2)
IOU_T = np.float32(0.5)
HUMAN = np.int32(1)
MAXH = np.int32(15)
MAXO = np.int32(15)
NOUT = 30

_GDN = lax.GatherDimensionNumbers(
    offset_dims=(), collapsed_slice_dims=(0,), start_index_map=(0,))


def _perm(x, idx):
    return lax.gather(x, idx[:, None], _GDN, (1,),
                      mode=lax.GatherScatterMode.PROMISE_IN_BOUNDS)


def _argmax_splat(m, mi, lanes):
    for k in (8, 4, 2, 1):
        pidx = lanes ^ k
        pm = _perm(m, pidx)
        pmi = _perm(mi, pidx)
        better = (pm > m) | ((pm == m) & (pmi < mi))
        m = jnp.where(better, pm, m)
        mi = jnp.where(better, pmi, mi)
    return m, mi


def _sc_nms(x1h, y1h, x2h, y2h, sh, labh, pack_o, olab_o, oval_o,
            x1v, y1v, x2v, y2v, sv, labv,
            ox1v, oy1v, ox2v, oy2v, keyv, fillv,
            pack_v, olab_v, oval_v, msc, misc, fmisc):
    is0 = (lax.axis_index("c") == 0) & (lax.axis_index("s") == 0)

    @pl.when(is0)
    def _():
        lanes = lax.iota(jnp.int32, 16)

        pltpu.sync_copy(x1h, x1v)
        pltpu.sync_copy(y1h, y1v)
        pltpu.sync_copy(x2h, x2v)
        pltpu.sync_copy(y2h, y2v)
        pltpu.sync_copy(sh, sv)
        pltpu.sync_copy(labh, labv)

        def s1(c, m):
            d = pl.ds(c * 16, 16)
            return jnp.maximum(m, jnp.maximum(x2v[d], y2v[d]))
        mcv = lax.fori_loop(0, NCHUNK, s1, jnp.full((16,), NEG, jnp.float32))
        for k in (8, 4, 2, 1):
            mcv = jnp.maximum(mcv, _perm(mcv, lanes ^ k))
        mcv = mcv + 1.0

        def s2(c, carry):
            m, mi = carry
            d = pl.ds(c * 16, 16)
            idxv = c * 16 + lanes
            off = labv[d].astype(jnp.float32) * mcv
            a = x1v[d] + off
            b = y1v[d] + off
            cc = x2v[d] + off
            dd = y2v[d] + off
            ox1v[d] = a
            oy1v[d] = b
            ox2v[d] = cc
            oy2v[d] = dd
            sc = sv[d]
            vmask = sc >= SCORE_T
            keyc = jnp.where(vmask, sc, NEG)
            keyv[d] = keyc
            idxf = idxv.astype(jnp.float32)
            fillv[d] = jnp.where(vmask, sc,
                                 jnp.where(idxv < N, -(idxf + 2.0), NEG))
            upd = keyc > m
            return jnp.where(upd, keyc, m), jnp.where(upd, idxv, mi)

        m0, mi0 = lax.fori_loop(
            0, NCHUNK, s2,
            (jnp.full((16,), NEG, jnp.float32), jnp.zeros((16,), jnp.int32)))
        m0, mi0 = _argmax_splat(m0, mi0, lanes)
        msc[...] = m0
        misc[...] = mi0
        fmisc[...] = jnp.zeros((16,), jnp.int32)

        def body(t, carry):
            h, o = carry
            gm = msc[...][0]
            gi = misc[...][0]
            active = gm > NEG_HALF

            @pl.when(jnp.logical_not(active))
            def _():
                def fsweep(c, fcarry):
                    m, mi = fcarry
                    fc = fillv[pl.ds(c * 16, 16)]
                    upd = fc > m
                    return (jnp.where(upd, fc, m),
                            jnp.where(upd, c * 16 + lanes, mi))
                fm, fmi = lax.fori_loop(
                    0, NCHUNK, fsweep,
                    (jnp.full((16,), NEG, jnp.float32),
                     jnp.zeros((16,), jnp.int32)))
                fm, fmi = _argmax_splat(fm, fmi, lanes)
                fmisc[...] = fmi

            pick = jnp.where(active, gi, fmisc[...][0])

            fd = pl.ds(pick, 16)
            fold = fillv[fd]
            fillv[fd] = jnp.where(lanes == 0, NEG, fold)

            cx1 = x1v[fd][0]
            cy1 = y1v[fd][0]
            cx2 = x2v[fd][0]
            cy2 = y2v[fd][0]
            cs = sv[fd][0]
            plab = labv[fd][0]

            is_h = plab == HUMAN
            inc = active.astype(jnp.int32)
            h2 = h + jnp.where(is_h, inc, 0)
            o2 = o + jnp.where(is_h, 0, inc)


            vals = jnp.where(lanes == 0, cx1,
                   jnp.where(lanes == 1, cy1,
                   jnp.where(lanes == 2, cx2,
                   jnp.where(lanes == 3, cy2, cs))))
            pd = pl.ds(t * 5, 16)
            pold = pack_v[pd]
            pack_v[pd] = jnp.where(lanes < 5, vals, pold)
            od = pl.ds(t, 16)
            lold = olab_v[od]
            olab_v[od] = jnp.where(lanes == 0, plab, lold)
            vold = oval_v[od]
            oval_v[od] = jnp.where(lanes == 0, inc, vold)

            @pl.when(active)
            def _():
                pox1 = ox1v[fd][0]
                poy1 = oy1v[fd][0]
                pox2 = ox2v[fd][0]
                poy2 = oy2v[fd][0]
                poarea = (pox2 - pox1) * (poy2 - poy1)

                ph = 1 - jnp.minimum(jnp.abs(plab - HUMAN), 1)
                capchg = jnp.where(ph == 1, h2 == MAXH, o2 == MAXO)

                @pl.when(capchg)
                def _():
                    phv = jnp.full((16,), ph, jnp.int32)

                    def ksweep(c, _):
                        d = pl.ds(c * 16, 16)
                        labc = labv[d]
                        labh = 1 - jnp.minimum(jnp.abs(labc - HUMAN), 1)
                        keyv[d] = jnp.where(labh == phv, NEG, keyv[d])
                        return 0

                    lax.fori_loop(0, NCHUNK, ksweep, 0)

                def sweep(c, scarry):
                    m, mi = scarry
                    for k in range(4):
                        d = pl.ds(c * 64 + k * 16, 16)
                        idxv = c * 64 + k * 16 + lanes
                        a = ox1v[d]
                        b = oy1v[d]
                        cc = ox2v[d]
                        dd = oy2v[d]
                        ltx = jnp.maximum(pox1, a)
                        lty = jnp.maximum(poy1, b)
                        rbx = jnp.minimum(pox2, cc)
                        rby = jnp.minimum(poy2, dd)
                        w = jnp.maximum(rbx - ltx, 0.0)
                        hh = jnp.maximum(rby - lty, 0.0)
                        inter = w * hh
                        union = poarea + (cc - a) * (dd - b) - inter
                        iou = inter / jnp.maximum(union, np.float32(1e-8))
                        keyc = jnp.where(iou > IOU_T, NEG, keyv[d])
                        keyv[d] = keyc
                        upd = keyc > m
                        m = jnp.where(upd, keyc, m)
                        mi = jnp.where(upd, idxv, mi)
                    return m, mi

                m, mi = lax.fori_loop(
                    0, NCHUNK // 4, sweep,
                    (jnp.full((16,), NEG, jnp.float32),
                     jnp.zeros((16,), jnp.int32)))
                m, mi = _argmax_splat(m, mi, lanes)
                msc[...] = m
                misc[...] = mi

            return h2, o2

        lax.fori_loop(0, NOUT, body, (jnp.int32(0), jnp.int32(0)))

        pltpu.sync_copy(pack_v, pack_o)
        pltpu.sync_copy(olab_v, olab_o)
        pltpu.sync_copy(oval_v, oval_o)


@jax.jit
def kernel(boxes, scores, labels):
    pad = NPAD - N
    x1 = jnp.pad(boxes[:, 0], (0, pad))
    y1 = jnp.pad(boxes[:, 1], (0, pad))
    x2 = jnp.pad(boxes[:, 2], (0, pad))
    y2 = jnp.pad(boxes[:, 3], (0, pad))
    sp = jnp.pad(scores, (0, pad))
    lp = jnp.pad(labels, (0, pad))

    mesh = plsc.VectorSubcoreMesh(core_axis_name="c", subcore_axis_name="s")
    f = pl.kernel(
        _sc_nms, mesh=mesh,
        out_type=[
            jax.ShapeDtypeStruct((176,), jnp.float32),
            jax.ShapeDtypeStruct((48,), jnp.int32),
            jax.ShapeDtypeStruct((48,), jnp.int32),
        ],
        scratch_types=(
            [pltpu.VMEM((NPAD,), jnp.float32) for _ in range(5)]
            + [pltpu.VMEM((NPAD,), jnp.int32)]
            + [pltpu.VMEM((NPAD,), jnp.float32) for _ in range(6)]
            + [pltpu.VMEM((176,), jnp.float32),
               pltpu.VMEM((48,), jnp.int32),
               pltpu.VMEM((48,), jnp.int32),
               pltpu.VMEM((16,), jnp.float32),
               pltpu.VMEM((16,), jnp.int32),
               pltpu.VMEM((16,), jnp.int32)]),
    )
    pack, olab, oval = f(x1, y1, x2, y2, sp, lp)
    packed = pack[:150].reshape(30, 5)
    return packed, olab[:30], oval[:30].astype(bool)


# speedup vs baseline: 4.4784x; 1.5858x over previous
import jax
import jax.numpy as jnp
import numpy as np
from jax import lax
from jax.experimental import pallas as pl
from jax.experimental.pallas import tpu as pltpu
from jax.experimental.pallas import tpu_sc as plsc

N = 5000
NPAD = 5120
NW = 16
OWN = NPAD // NW
OWNC = OWN // 16
LPAD = OWN + 16
NEG = np.float32(-1e30)
NEG_HALF = np.float32(-5e29)
BLO = np.float32(-3e38)
SCORE_T = np.float32(0.2)
IOU_T = np.float32(0.5)
HUMAN = np.int32(1)
MAXH = np.int32(15)
MAXO = np.int32(15)
NOUT = 30

_GDN = lax.GatherDimensionNumbers(
    offset_dims=(), collapsed_slice_dims=(0,), start_index_map=(0,))


def _perm(x, idx):
    return lax.gather(x, idx[:, None], _GDN, (1,),
                      mode=lax.GatherScatterMode.PROMISE_IN_BOUNDS)


def _argmax_splat(m, mi, lanes):
    for k in (8, 4, 2, 1):
        pidx = lanes ^ k
        pm = _perm(m, pidx)
        pmi = _perm(mi, pidx)
        better = (pm > m) | ((pm == m) & (pmi < mi))
        m = jnp.where(better, pm, m)
        mi = jnp.where(better, pmi, mi)
    return m, mi


def _sc_nms(x1h, y1h, x2h, y2h, sh, labh, pack_o, olab_o, oval_o,
            lx1, ly1, lx2, ly2, ls, llab,
            lox1, loy1, lox2, loy2, lkey, lfill,
            pack_v, olab_v, oval_v, rowbuf, puball, pub_sh):
    cid = lax.axis_index("c")
    sid = lax.axis_index("s")
    lanes = lax.iota(jnp.int32, 16)
    base = sid * OWN
    is_out = (cid == 0) & (sid == 0)

    pltpu.sync_copy(x1h.at[pl.ds(base, OWN)], lx1.at[pl.ds(0, OWN)])
    pltpu.sync_copy(y1h.at[pl.ds(base, OWN)], ly1.at[pl.ds(0, OWN)])
    pltpu.sync_copy(x2h.at[pl.ds(base, OWN)], lx2.at[pl.ds(0, OWN)])
    pltpu.sync_copy(y2h.at[pl.ds(base, OWN)], ly2.at[pl.ds(0, OWN)])
    pltpu.sync_copy(sh.at[pl.ds(base, OWN)], ls.at[pl.ds(0, OWN)])
    pltpu.sync_copy(labh.at[pl.ds(base, OWN)], llab.at[pl.ds(0, OWN)])

    def publish_round():
        pltpu.sync_copy(rowbuf, pub_sh.at[pl.ds(sid * 16, 16)])
        plsc.subcore_barrier()
        pltpu.sync_copy(pub_sh, puball)
        plsc.subcore_barrier()

    def reduce_rows():
        bm = BLO
        bi = np.float32(3e38)
        brow = jnp.zeros((16,), jnp.float32)
        for r in range(NW):
            rrow = puball[pl.ds(r * 16, 16)]
            mr = rrow[0]
            ir = rrow[1]
            better = (mr > bm) | ((mr == bm) & (ir < bi))
            bf = jnp.where(better, np.float32(1.0), np.float32(0.0))
            bfv = jnp.full((16,), bf, jnp.float32)
            brow = bfv * rrow + (1.0 - bfv) * brow
            bm = jnp.where(better, mr, bm)
            bi = jnp.where(better, ir, bi)
        return brow

    def mk_row(m, mi, with_coords):
        ms = m[0]
        gi = mi[0]
        loff = jnp.clip(gi - base, 0, OWN - 1)
        fd = pl.ds(loff, 16)
        if with_coords:
            coords = (lox1[fd][0], loy1[fd][0], lox2[fd][0], loy2[fd][0])
        else:
            z = np.float32(0.0)
            coords = (z, z, z, z)
        vals = (ms, gi.astype(jnp.float32)) + coords + (
            lx1[fd][0], ly1[fd][0], lx2[fd][0], ly2[fd][0],
            ls[fd][0], llab[fd][0].astype(jnp.float32))
        row = jnp.full((16,), np.float32(0.0), jnp.float32)
        for j, v in enumerate(vals):
            row = jnp.where(lanes == j, v, row)
        return row

    def s1(c, m):
        d = pl.ds(c * 16, 16)
        return jnp.maximum(m, jnp.maximum(lx2[d], ly2[d]))
    mloc = lax.fori_loop(0, OWNC, s1, jnp.full((16,), NEG, jnp.float32))
    for k in (8, 4, 2, 1):
        mloc = jnp.maximum(mloc, _perm(mloc, lanes ^ k))
    rowbuf[...] = jnp.where(lanes == 0, mloc[0], np.float32(0.0))
    publish_round()
    mc = BLO
    for r in range(NW):
        mc = jnp.maximum(mc, puball[pl.ds(r * 16, 16)][0])
    mc = mc + 1.0

    def s2(c, carry):
        m, mi = carry
        d = pl.ds(c * 16, 16)
        idxv = base + c * 16 + lanes
        off = llab[d].astype(jnp.float32) * mc
        a = lx1[d] + off
        b = ly1[d] + off
        cc = lx2[d] + off
        dd = ly2[d] + off
        lox1[d] = a
        loy1[d] = b
        lox2[d] = cc
        loy2[d] = dd
        sc = ls[d]
        vmask = sc >= SCORE_T
        keyc = jnp.where(vmask, sc, NEG)
        lkey[d] = keyc
        idxf = idxv.astype(jnp.float32)
        lfill[d] = jnp.where(vmask, sc,
                             jnp.where(idxv < N, -(idxf + 2.0), NEG))
        upd = keyc > m
        return jnp.where(upd, keyc, m), jnp.where(upd, idxv, mi)

    m0, mi0 = lax.fori_loop(
        0, OWNC, s2,
        (jnp.full((16,), NEG, jnp.float32), jnp.zeros((16,), jnp.int32)))
    m0, mi0 = _argmax_splat(m0, mi0, lanes)
    rowbuf[...] = mk_row(m0, mi0, True)
    publish_round()

    def body(t, carry):
        h, o = carry
        brow = reduce_rows()
        active = brow[0] > NEG_HALF

        @pl.when(jnp.logical_not(active))
        def _():
            def fsweep(c, fcarry):
                m, mi = fcarry
                fc = lfill[pl.ds(c * 16, 16)]
                idxv = base + c * 16 + lanes
                upd = fc > m
                return jnp.where(upd, fc, m), jnp.where(upd, idxv, mi)
            fm, fmi = lax.fori_loop(
                0, OWNC, fsweep,
                (jnp.full((16,), NEG, jnp.float32),
                 jnp.zeros((16,), jnp.int32)))
            fm, fmi = _argmax_splat(fm, fmi, lanes)
            rowbuf[...] = mk_row(fm, fmi, False)
            publish_round()

        brow2 = reduce_rows()
        pick = brow2[1].astype(jnp.int32)
        plab = brow2[11].astype(jnp.int32)

        is_h = plab == HUMAN
        inc = active.astype(jnp.int32)
        h2 = h + jnp.where(is_h, inc, 0)
        o2 = o + jnp.where(is_h, 0, inc)

        loffp = jnp.clip(pick - base, 0, OWN - 1)
        inr = (pick >= base) & (pick < base + OWN)
        kd = pl.ds(loffp, 16)
        fold = lfill[kd]
        v0 = jnp.where(inr, NEG, fold[0])
        lfill[kd] = jnp.where(lanes == 0, v0, fold)

        @pl.when(is_out)
        def _():
            vals = jnp.where(lanes == 0, brow2[6],
                   jnp.where(lanes == 1, brow2[7],
                   jnp.where(lanes == 2, brow2[8],
                   jnp.where(lanes == 3, brow2[9], brow2[10]))))
            pd = pl.ds(t * 5, 16)
            pold = pack_v[pd]
            pack_v[pd] = jnp.where(lanes < 5, vals, pold)
            od = pl.ds(t, 16)
            lold = olab_v[od]
            olab_v[od] = jnp.where(lanes == 0, plab, lold)
            vold = oval_v[od]
            oval_v[od] = jnp.where(lanes == 0, inc, vold)

        rowbuf[...] = jnp.where(lanes == 0, NEG, np.float32(0.0))

        @pl.when(active)
        def _():
            pox1 = brow2[2]
            poy1 = brow2[3]
            pox2 = brow2[4]
            poy2 = brow2[5]
            poarea = (pox2 - pox1) * (poy2 - poy1)

            ph = 1 - jnp.minimum(jnp.abs(plab - HUMAN), 1)
            capchg = jnp.where(ph == 1, h2 == MAXH, o2 == MAXO)

            @pl.when(capchg)
            def _():
                phv = jnp.full((16,), ph, jnp.int32)

                def ksweep(c, _):
                    d = pl.ds(c * 16, 16)
                    labc = llab[d]
                    labhum = 1 - jnp.minimum(jnp.abs(labc - HUMAN), 1)
                    lkey[d] = jnp.where(labhum == phv, NEG, lkey[d])
                    return 0

                lax.fori_loop(0, OWNC, ksweep, 0)

            def sweep(c, scarry):
                m, mi = scarry
                for k in range(4):
                    d = pl.ds(c * 64 + k * 16, 16)
                    idxv = base + c * 64 + k * 16 + lanes
                    a = lox1[d]
                    b = loy1[d]
                    cc = lox2[d]
                    dd = loy2[d]
                    ltx = jnp.maximum(pox1, a)
                    lty = jnp.maximum(poy1, b)
                    rbx = jnp.minimum(pox2, cc)
                    rby = jnp.minimum(poy2, dd)
                    w = jnp.maximum(rbx - ltx, 0.0)
                    hh = jnp.maximum(rby - lty, 0.0)
                    inter = w * hh
                    union = poarea + (cc - a) * (dd - b) - inter
                    iou = inter / jnp.maximum(union, np.float32(1e-8))
                    keyc = jnp.where(iou > IOU_T, NEG, lkey[d])
                    lkey[d] = keyc
                    upd = keyc > m
                    m = jnp.where(upd, keyc, m)
                    mi = jnp.where(upd, idxv, mi)
                return m, mi

            m, mi = lax.fori_loop(
                0, OWNC // 4, sweep,
                (jnp.full((16,), NEG, jnp.float32),
                 jnp.zeros((16,), jnp.int32)))
            m, mi = _argmax_splat(m, mi, lanes)
            rowbuf[...] = mk_row(m, mi, True)

        publish_round()
        return h2, o2

    lax.fori_loop(0, NOUT, body, (jnp.int32(0), jnp.int32(0)))

    @pl.when(is_out)
    def _():
        pltpu.sync_copy(pack_v, pack_o)
        pltpu.sync_copy(olab_v, olab_o)
        pltpu.sync_copy(oval_v, oval_o)


@jax.jit
def kernel(boxes, scores, labels):
    pad = NPAD - N
    x1 = jnp.pad(boxes[:, 0], (0, pad))
    y1 = jnp.pad(boxes[:, 1], (0, pad))
    x2 = jnp.pad(boxes[:, 2], (0, pad))
    y2 = jnp.pad(boxes[:, 3], (0, pad))
    sp = jnp.pad(scores, (0, pad))
    lp = jnp.pad(labels, (0, pad))

    mesh = plsc.VectorSubcoreMesh(core_axis_name="c", subcore_axis_name="s")
    f = pl.kernel(
        _sc_nms, mesh=mesh,
        out_type=[
            jax.ShapeDtypeStruct((176,), jnp.float32),
            jax.ShapeDtypeStruct((48,), jnp.int32),
            jax.ShapeDtypeStruct((48,), jnp.int32),
        ],
        scratch_types=(
            [pltpu.VMEM((LPAD,), jnp.float32) for _ in range(5)]
            + [pltpu.VMEM((LPAD,), jnp.int32)]
            + [pltpu.VMEM((LPAD,), jnp.float32) for _ in range(6)]
            + [pltpu.VMEM((176,), jnp.float32),
               pltpu.VMEM((48,), jnp.int32),
               pltpu.VMEM((48,), jnp.int32),
               pltpu.VMEM((16,), jnp.float32),
               pltpu.VMEM((NW * 16,), jnp.float32),
               pltpu.VMEM_SHARED((NW * 16,), jnp.float32)]),
    )
    pack, olab, oval = f(x1, y1, x2, y2, sp, lp)
    packed = pack[:150].reshape(30, 5)
    return packed, olab[:30], oval[:30].astype(bool)


# speedup vs baseline: 4.7592x; 1.0627x over previous
import jax
import jax.numpy as jnp
import numpy as np
from jax import lax
from jax.experimental import pallas as pl
from jax.experimental.pallas import tpu as pltpu
from jax.experimental.pallas import tpu_sc as plsc

N = 5000
NPAD = 5120
NW = 16
OWN = NPAD // NW
OWNC = OWN // 16
LPAD = OWN + 16
NEG = np.float32(-1e30)
NEG_HALF = np.float32(-5e29)
BLO = np.float32(-3e38)
SCORE_T = np.float32(0.2)
IOU_T = np.float32(0.5)
HUMAN = np.int32(1)
MAXH = np.int32(15)
MAXO = np.int32(15)
NOUT = 30

_GDN = lax.GatherDimensionNumbers(
    offset_dims=(), collapsed_slice_dims=(0,), start_index_map=(0,))


def _perm(x, idx):
    return lax.gather(x, idx[:, None], _GDN, (1,),
                      mode=lax.GatherScatterMode.PROMISE_IN_BOUNDS)


def _argmax_splat(m, mi, lanes):
    for k in (8, 4, 2, 1):
        pidx = lanes ^ k
        pm = _perm(m, pidx)
        pmi = _perm(mi, pidx)
        better = (pm > m) | ((pm == m) & (pmi < mi))
        m = jnp.where(better, pm, m)
        mi = jnp.where(better, pmi, mi)
    return m, mi


def _sc_nms(x1h, y1h, x2h, y2h, sh, labh, pack_o, olab_o, oval_o,
            lx1, ly1, lx2, ly2, ls, llab,
            lox1, loy1, lox2, loy2, lkey, lfill,
            pack_v, olab_v, oval_v, rowbuf, puball, pub_sh):
    cid = lax.axis_index("c")
    sid = lax.axis_index("s")
    lanes = lax.iota(jnp.int32, 16)
    base = sid * OWN
    is_out = (cid == 0) & (sid == 0)

    pltpu.sync_copy(x1h.at[pl.ds(base, OWN)], lx1.at[pl.ds(0, OWN)])
    pltpu.sync_copy(y1h.at[pl.ds(base, OWN)], ly1.at[pl.ds(0, OWN)])
    pltpu.sync_copy(x2h.at[pl.ds(base, OWN)], lx2.at[pl.ds(0, OWN)])
    pltpu.sync_copy(y2h.at[pl.ds(base, OWN)], ly2.at[pl.ds(0, OWN)])
    pltpu.sync_copy(sh.at[pl.ds(base, OWN)], ls.at[pl.ds(0, OWN)])
    pltpu.sync_copy(labh.at[pl.ds(base, OWN)], llab.at[pl.ds(0, OWN)])

    def publish_round():
        pltpu.sync_copy(rowbuf, pub_sh.at[pl.ds(sid * 16, 16)])
        plsc.subcore_barrier()
        pltpu.sync_copy(pub_sh, puball)
        plsc.subcore_barrier()

    def reduce_rows():
        bm = BLO
        bi = np.float32(3e38)
        brow = jnp.zeros((16,), jnp.float32)
        for r in range(NW):
            rrow = puball[pl.ds(r * 16, 16)]
            mr = rrow[0]
            ir = rrow[1]
            better = (mr > bm) | ((mr == bm) & (ir < bi))
            bf = jnp.where(better, np.float32(1.0), np.float32(0.0))
            bfv = jnp.full((16,), bf, jnp.float32)
            brow = bfv * rrow + (1.0 - bfv) * brow
            bm = jnp.where(better, mr, bm)
            bi = jnp.where(better, ir, bi)
        return brow

    def mk_row(m, mi, with_coords):
        ms = m[0]
        gi = mi[0]
        loff = jnp.clip(gi - base, 0, OWN - 1)
        fd = pl.ds(loff, 16)
        if with_coords:
            coords = (lox1[fd][0], loy1[fd][0], lox2[fd][0], loy2[fd][0])
        else:
            z = np.float32(0.0)
            coords = (z, z, z, z)
        vals = (ms, gi.astype(jnp.float32)) + coords + (
            lx1[fd][0], ly1[fd][0], lx2[fd][0], ly2[fd][0],
            ls[fd][0], llab[fd][0].astype(jnp.float32))
        row = jnp.full((16,), np.float32(0.0), jnp.float32)
        for j, v in enumerate(vals):
            row = jnp.where(lanes == j, v, row)
        return row

    def s1(c, m):
        d = pl.ds(c * 16, 16)
        return jnp.maximum(m, jnp.maximum(lx2[d], ly2[d]))
    mloc = lax.fori_loop(0, OWNC, s1, jnp.full((16,), NEG, jnp.float32))
    for k in (8, 4, 2, 1):
        mloc = jnp.maximum(mloc, _perm(mloc, lanes ^ k))
    rowbuf[...] = jnp.where(lanes == 0, mloc[0], np.float32(0.0))
    publish_round()
    mc = BLO
    for r in range(NW):
        mc = jnp.maximum(mc, puball[pl.ds(r * 16, 16)][0])
    mc = mc + 1.0

    def s2(c, carry):
        m, mi = carry
        d = pl.ds(c * 16, 16)
        idxv = base + c * 16 + lanes
        off = llab[d].astype(jnp.float32) * mc
        a = lx1[d] + off
        b = ly1[d] + off
        cc = lx2[d] + off
        dd = ly2[d] + off
        lox1[d] = a
        loy1[d] = b
        lox2[d] = cc
        loy2[d] = dd
        sc = ls[d]
        vmask = sc >= SCORE_T
        keyc = jnp.where(vmask, sc, NEG)
        lkey[d] = keyc
        idxf = idxv.astype(jnp.float32)
        lfill[d] = jnp.where(vmask, sc,
                             jnp.where(idxv < N, -(idxf + 2.0), NEG))
        upd = keyc > m
        return jnp.where(upd, keyc, m), jnp.where(upd, idxv, mi)

    m0, mi0 = lax.fori_loop(
        0, OWNC, s2,
        (jnp.full((16,), NEG, jnp.float32), jnp.zeros((16,), jnp.int32)))
    m0, mi0 = _argmax_splat(m0, mi0, lanes)
    rowbuf[...] = mk_row(m0, mi0, True)
    publish_round()

    def body(t, carry):
        h, o = carry
        brow = reduce_rows()
        active = brow[0] > NEG_HALF

        @pl.when(jnp.logical_not(active))
        def _():
            def fsweep(c, fcarry):
                m, mi = fcarry
                fc = lfill[pl.ds(c * 16, 16)]
                idxv = base + c * 16 + lanes
                upd = fc > m
                return jnp.where(upd, fc, m), jnp.where(upd, idxv, mi)
            fm, fmi = lax.fori_loop(
                0, OWNC, fsweep,
                (jnp.full((16,), NEG, jnp.float32),
                 jnp.zeros((16,), jnp.int32)))
            fm, fmi = _argmax_splat(fm, fmi, lanes)
            rowbuf[...] = mk_row(fm, fmi, False)
            publish_round()

        brow2 = reduce_rows()
        pick = brow2[1].astype(jnp.int32)
        plab = brow2[11].astype(jnp.int32)

        is_h = plab == HUMAN
        inc = active.astype(jnp.int32)
        h2 = h + jnp.where(is_h, inc, 0)
        o2 = o + jnp.where(is_h, 0, inc)

        loffp = jnp.clip(pick - base, 0, OWN - 1)
        inr = (pick >= base) & (pick < base + OWN)
        kd = pl.ds(loffp, 16)
        fold = lfill[kd]
        v0 = jnp.where(inr, NEG, fold[0])
        lfill[kd] = jnp.where(lanes == 0, v0, fold)

        @pl.when(is_out)
        def _():
            vals = jnp.where(lanes == 0, brow2[6],
                   jnp.where(lanes == 1, brow2[7],
                   jnp.where(lanes == 2, brow2[8],
                   jnp.where(lanes == 3, brow2[9], brow2[10]))))
            pd = pl.ds(t * 5, 16)
            pold = pack_v[pd]
            pack_v[pd] = jnp.where(lanes < 5, vals, pold)
            od = pl.ds(t, 16)
            lold = olab_v[od]
            olab_v[od] = jnp.where(lanes == 0, plab, lold)
            vold = oval_v[od]
            oval_v[od] = jnp.where(lanes == 0, inc, vold)

        rowbuf[...] = jnp.where(lanes == 0, NEG, np.float32(0.0))

        @pl.when(active)
        def _():
            pox1 = brow2[2]
            poy1 = brow2[3]
            pox2 = brow2[4]
            poy2 = brow2[5]
            poarea = (pox2 - pox1) * (poy2 - poy1)

            ph = 1 - jnp.minimum(jnp.abs(plab - HUMAN), 1)
            capchg = jnp.where(ph == 1, h2 == MAXH, o2 == MAXO)

            @pl.when(capchg)
            def _():
                phv = jnp.full((16,), ph, jnp.int32)

                def ksweep(c, _):
                    d = pl.ds(c * 16, 16)
                    labc = llab[d]
                    labhum = 1 - jnp.minimum(jnp.abs(labc - HUMAN), 1)
                    lkey[d] = jnp.where(labhum == phv, NEG, lkey[d])
                    return 0

                lax.fori_loop(0, OWNC, ksweep, 0)

            def sweep(c, scarry):
                m, mi = scarry
                for k in range(4):
                    d = pl.ds(c * 64 + k * 16, 16)
                    idxv = base + c * 64 + k * 16 + lanes
                    a = lox1[d]
                    b = loy1[d]
                    cc = lox2[d]
                    dd = loy2[d]
                    ltx = jnp.maximum(pox1, a)
                    lty = jnp.maximum(poy1, b)
                    rbx = jnp.minimum(pox2, cc)
                    rby = jnp.minimum(poy2, dd)
                    w = jnp.maximum(rbx - ltx, 0.0)
                    hh = jnp.maximum(rby - lty, 0.0)
                    inter = w * hh
                    union = poarea + (cc - a) * (dd - b) - inter
                    iou = inter / jnp.maximum(union, np.float32(1e-8))
                    keyc = jnp.where(iou > IOU_T, NEG, lkey[d])
                    lkey[d] = keyc
                    upd = keyc > m
                    m = jnp.where(upd, keyc, m)
                    mi = jnp.where(upd, idxv, mi)
                return m, mi

            m, mi = lax.fori_loop(
                0, OWNC // 4, sweep,
                (jnp.full((16,), NEG, jnp.float32),
                 jnp.zeros((16,), jnp.int32)))
            m, mi = _argmax_splat(m, mi, lanes)
            rowbuf[...] = mk_row(m, mi, True)

        publish_round()
        return h2, o2

    lax.fori_loop(0, NOUT, body, (jnp.int32(0), jnp.int32(0)))

    @pl.when(is_out)
    def _():
        pltpu.sync_copy(pack_v, pack_o)
        pltpu.sync_copy(olab_v, olab_o)
        pltpu.sync_copy(oval_v, oval_o)


@jax.jit
def kernel(boxes, scores, labels):
    pad = NPAD - N
    x1 = jnp.pad(boxes[:, 0], (0, pad))
    y1 = jnp.pad(boxes[:, 1], (0, pad))
    x2 = jnp.pad(boxes[:, 2], (0, pad))
    y2 = jnp.pad(boxes[:, 3], (0, pad))
    sp = jnp.pad(scores, (0, pad))
    lp = jnp.pad(labels, (0, pad))

    mesh = plsc.VectorSubcoreMesh(core_axis_name="c", subcore_axis_name="s", num_cores=1)
    f = pl.kernel(
        _sc_nms, mesh=mesh,
        out_type=[
            jax.ShapeDtypeStruct((176,), jnp.float32),
            jax.ShapeDtypeStruct((48,), jnp.int32),
            jax.ShapeDtypeStruct((48,), jnp.int32),
        ],
        scratch_types=(
            [pltpu.VMEM((LPAD,), jnp.float32) for _ in range(5)]
            + [pltpu.VMEM((LPAD,), jnp.int32)]
            + [pltpu.VMEM((LPAD,), jnp.float32) for _ in range(6)]
            + [pltpu.VMEM((176,), jnp.float32),
               pltpu.VMEM((48,), jnp.int32),
               pltpu.VMEM((48,), jnp.int32),
               pltpu.VMEM((16,), jnp.float32),
               pltpu.VMEM((NW * 16,), jnp.float32),
               pltpu.VMEM_SHARED((NW * 16,), jnp.float32)]),
    )
    pack, olab, oval = f(x1, y1, x2, y2, sp, lp)
    packed = pack[:150].reshape(30, 5)
    return packed, olab[:30], oval[:30].astype(bool)


# speedup vs baseline: 4.9705x; 1.0444x over previous
import jax
import jax.numpy as jnp
import numpy as np
from jax import lax
from jax.experimental import pallas as pl
from jax.experimental.pallas import tpu as pltpu
from jax.experimental.pallas import tpu_sc as plsc

N = 5000
NPAD = 5120
NW = 16
OWN = NPAD // NW
OWNC = OWN // 16
LPAD = OWN + 16
NEG = np.float32(-1e30)
NEG_HALF = np.float32(-5e29)
BLO = np.float32(-3e38)
SCORE_T = np.float32(0.2)
IOU_T = np.float32(0.5)
HUMAN = np.int32(1)
MAXH = np.int32(15)
MAXO = np.int32(15)
NOUT = 30

_GDN = lax.GatherDimensionNumbers(
    offset_dims=(), collapsed_slice_dims=(0,), start_index_map=(0,))


def _perm(x, idx):
    return lax.gather(x, idx[:, None], _GDN, (1,),
                      mode=lax.GatherScatterMode.PROMISE_IN_BOUNDS)


def _argmax_splat(m, mi, lanes):
    for k in (8, 4, 2, 1):
        pidx = lanes ^ k
        pm = _perm(m, pidx)
        pmi = _perm(mi, pidx)
        better = (pm > m) | ((pm == m) & (pmi < mi))
        m = jnp.where(better, pm, m)
        mi = jnp.where(better, pmi, mi)
    return m, mi


def _sc_nms(x1h, y1h, x2h, y2h, sh, labh, pack_o, olab_o, oval_o,
            lx1, ly1, lx2, ly2, ls, llab,
            lox1, loy1, lox2, loy2, lkey, lfill,
            pack_v, olab_v, oval_v, rowbuf, fbrow, puball, pub_sh):
    cid = lax.axis_index("c")
    sid = lax.axis_index("s")
    lanes = lax.iota(jnp.int32, 16)
    base = sid * OWN
    is_out = (cid == 0) & (sid == 0)

    pltpu.sync_copy(x1h.at[pl.ds(base, OWN)], lx1.at[pl.ds(0, OWN)])
    pltpu.sync_copy(y1h.at[pl.ds(base, OWN)], ly1.at[pl.ds(0, OWN)])
    pltpu.sync_copy(x2h.at[pl.ds(base, OWN)], lx2.at[pl.ds(0, OWN)])
    pltpu.sync_copy(y2h.at[pl.ds(base, OWN)], ly2.at[pl.ds(0, OWN)])
    pltpu.sync_copy(sh.at[pl.ds(base, OWN)], ls.at[pl.ds(0, OWN)])
    pltpu.sync_copy(labh.at[pl.ds(base, OWN)], llab.at[pl.ds(0, OWN)])

    def publish_round(off):
        pltpu.sync_copy(rowbuf, pub_sh.at[pl.ds(off + sid * 16, 16)])
        plsc.subcore_barrier()
        pltpu.sync_copy(pub_sh.at[pl.ds(off, NW * 16)], puball)

    def reduce_rows():
        bm = BLO
        bi = np.float32(3e38)
        brow = jnp.zeros((16,), jnp.float32)
        for r in range(NW):
            rrow = puball[pl.ds(r * 16, 16)]
            mr = rrow[0]
            ir = rrow[1]
            better = (mr > bm) | ((mr == bm) & (ir < bi))
            bf = jnp.where(better, np.float32(1.0), np.float32(0.0))
            bfv = jnp.full((16,), bf, jnp.float32)
            brow = bfv * rrow + (1.0 - bfv) * brow
            bm = jnp.where(better, mr, bm)
            bi = jnp.where(better, ir, bi)
        return brow

    def mk_row(m, mi, with_coords):
        ms = m[0]
        gi = mi[0]
        loff = jnp.clip(gi - base, 0, OWN - 1)
        fd = pl.ds(loff, 16)
        if with_coords:
            coords = (lox1[fd][0], loy1[fd][0], lox2[fd][0], loy2[fd][0])
        else:
            z = np.float32(0.0)
            coords = (z, z, z, z)
        vals = (ms, gi.astype(jnp.float32)) + coords + (
            lx1[fd][0], ly1[fd][0], lx2[fd][0], ly2[fd][0],
            ls[fd][0], llab[fd][0].astype(jnp.float32))
        row = jnp.full((16,), np.float32(0.0), jnp.float32)
        for j, v in enumerate(vals):
            row = jnp.where(lanes == j, v, row)
        return row

    def s1(c, m):
        d = pl.ds(c * 16, 16)
        return jnp.maximum(m, jnp.maximum(lx2[d], ly2[d]))
    mloc = lax.fori_loop(0, OWNC, s1, jnp.full((16,), NEG, jnp.float32))
    for k in (8, 4, 2, 1):
        mloc = jnp.maximum(mloc, _perm(mloc, lanes ^ k))
    rowbuf[...] = jnp.where(lanes == 0, mloc[0], np.float32(0.0))
    publish_round(2 * NW * 16)
    mc = BLO
    for r in range(NW):
        mc = jnp.maximum(mc, puball[pl.ds(r * 16, 16)][0])
    mc = mc + 1.0

    def s2(c, carry):
        m, mi = carry
        d = pl.ds(c * 16, 16)
        idxv = base + c * 16 + lanes
        off = llab[d].astype(jnp.float32) * mc
        a = lx1[d] + off
        b = ly1[d] + off
        cc = lx2[d] + off
        dd = ly2[d] + off
        lox1[d] = a
        loy1[d] = b
        lox2[d] = cc
        loy2[d] = dd
        sc = ls[d]
        vmask = sc >= SCORE_T
        keyc = jnp.where(vmask, sc, NEG)
        lkey[d] = keyc
        idxf = idxv.astype(jnp.float32)
        lfill[d] = jnp.where(vmask, sc,
                             jnp.where(idxv < N, -(idxf + 2.0), NEG))
        upd = keyc > m
        return jnp.where(upd, keyc, m), jnp.where(upd, idxv, mi)

    m0, mi0 = lax.fori_loop(
        0, OWNC, s2,
        (jnp.full((16,), NEG, jnp.float32), jnp.zeros((16,), jnp.int32)))
    m0, mi0 = _argmax_splat(m0, mi0, lanes)
    rowbuf[...] = mk_row(m0, mi0, True)
    fbrow[...] = jnp.zeros((16,), jnp.float32)
    publish_round(NW * 16)

    def body(t, carry):
        h, o = carry
        brow = reduce_rows()
        active = brow[0] > NEG_HALF

        @pl.when(jnp.logical_not(active))
        def _():
            def fsweep(c, fcarry):
                m, mi = fcarry
                fc = lfill[pl.ds(c * 16, 16)]
                idxv = base + c * 16 + lanes
                upd = fc > m
                return jnp.where(upd, fc, m), jnp.where(upd, idxv, mi)
            fm, fmi = lax.fori_loop(
                0, OWNC, fsweep,
                (jnp.full((16,), NEG, jnp.float32),
                 jnp.zeros((16,), jnp.int32)))
            fm, fmi = _argmax_splat(fm, fmi, lanes)
            rowbuf[...] = mk_row(fm, fmi, False)
            publish_round(2 * NW * 16)
            fbrow[...] = reduce_rows()

        af = jnp.where(active, np.float32(1.0), np.float32(0.0))
        afv = jnp.full((16,), af, jnp.float32)
        brow2 = afv * brow + (1.0 - afv) * fbrow[...]
        pick = brow2[1].astype(jnp.int32)
        plab = brow2[11].astype(jnp.int32)

        is_h = plab == HUMAN
        inc = active.astype(jnp.int32)
        h2 = h + jnp.where(is_h, inc, 0)
        o2 = o + jnp.where(is_h, 0, inc)

        loffp = jnp.clip(pick - base, 0, OWN - 1)
        inr = (pick >= base) & (pick < base + OWN)
        kd = pl.ds(loffp, 16)
        fold = lfill[kd]
        v0 = jnp.where(inr, NEG, fold[0])
        lfill[kd] = jnp.where(lanes == 0, v0, fold)

        @pl.when(is_out)
        def _():
            vals = jnp.where(lanes == 0, brow2[6],
                   jnp.where(lanes == 1, brow2[7],
                   jnp.where(lanes == 2, brow2[8],
                   jnp.where(lanes == 3, brow2[9], brow2[10]))))
            pd = pl.ds(t * 5, 16)
            pold = pack_v[pd]
            pack_v[pd] = jnp.where(lanes < 5, vals, pold)
            od = pl.ds(t, 16)
            lold = olab_v[od]
            olab_v[od] = jnp.where(lanes == 0, plab, lold)
            vold = oval_v[od]
            oval_v[od] = jnp.where(lanes == 0, inc, vold)

        rowbuf[...] = jnp.where(lanes == 0, NEG, np.float32(0.0))

        @pl.when(active)
        def _():
            pox1 = brow2[2]
            poy1 = brow2[3]
            pox2 = brow2[4]
            poy2 = brow2[5]
            poarea = (pox2 - pox1) * (poy2 - poy1)

            ph = 1 - jnp.minimum(jnp.abs(plab - HUMAN), 1)
            capchg = jnp.where(ph == 1, h2 == MAXH, o2 == MAXO)

            @pl.when(capchg)
            def _():
                phv = jnp.full((16,), ph, jnp.int32)

                def ksweep(c, _):
                    d = pl.ds(c * 16, 16)
                    labc = llab[d]
                    labhum = 1 - jnp.minimum(jnp.abs(labc - HUMAN), 1)
                    lkey[d] = jnp.where(labhum == phv, NEG, lkey[d])
                    return 0

                lax.fori_loop(0, OWNC, ksweep, 0)

            def sweep(c, scarry):
                m, mi = scarry
                for k in range(4):
                    d = pl.ds(c * 64 + k * 16, 16)
                    idxv = base + c * 64 + k * 16 + lanes
                    a = lox1[d]
                    b = loy1[d]
                    cc = lox2[d]
                    dd = loy2[d]
                    ltx = jnp.maximum(pox1, a)
                    lty = jnp.maximum(poy1, b)
                    rbx = jnp.minimum(pox2, cc)
                    rby = jnp.minimum(poy2, dd)
                    w = jnp.maximum(rbx - ltx, 0.0)
                    hh = jnp.maximum(rby - lty, 0.0)
                    inter = w * hh
                    union = poarea + (cc - a) * (dd - b) - inter
                    iou = inter / jnp.maximum(union, np.float32(1e-8))
                    keyc = jnp.where(iou > IOU_T, NEG, lkey[d])
                    lkey[d] = keyc
                    upd = keyc > m
                    m = jnp.where(upd, keyc, m)
                    mi = jnp.where(upd, idxv, mi)
                return m, mi

            m, mi = lax.fori_loop(
                0, OWNC // 4, sweep,
                (jnp.full((16,), NEG, jnp.float32),
                 jnp.zeros((16,), jnp.int32)))
            m, mi = _argmax_splat(m, mi, lanes)
            rowbuf[...] = mk_row(m, mi, True)

        publish_round((t % 2) * (NW * 16))
        return h2, o2

    lax.fori_loop(0, NOUT, body, (jnp.int32(0), jnp.int32(0)))

    @pl.when(is_out)
    def _():
        pltpu.sync_copy(pack_v, pack_o)
        pltpu.sync_copy(olab_v, olab_o)
        pltpu.sync_copy(oval_v, oval_o)


@jax.jit
def kernel(boxes, scores, labels):
    pad = NPAD - N
    x1 = jnp.pad(boxes[:, 0], (0, pad))
    y1 = jnp.pad(boxes[:, 1], (0, pad))
    x2 = jnp.pad(boxes[:, 2], (0, pad))
    y2 = jnp.pad(boxes[:, 3], (0, pad))
    sp = jnp.pad(scores, (0, pad))
    lp = jnp.pad(labels, (0, pad))

    mesh = plsc.VectorSubcoreMesh(core_axis_name="c", subcore_axis_name="s", num_cores=1)
    f = pl.kernel(
        _sc_nms, mesh=mesh,
        out_type=[
            jax.ShapeDtypeStruct((176,), jnp.float32),
            jax.ShapeDtypeStruct((48,), jnp.int32),
            jax.ShapeDtypeStruct((48,), jnp.int32),
        ],
        scratch_types=(
            [pltpu.VMEM((LPAD,), jnp.float32) for _ in range(5)]
            + [pltpu.VMEM((LPAD,), jnp.int32)]
            + [pltpu.VMEM((LPAD,), jnp.float32) for _ in range(6)]
            + [pltpu.VMEM((176,), jnp.float32),
               pltpu.VMEM((48,), jnp.int32),
               pltpu.VMEM((48,), jnp.int32),
               pltpu.VMEM((16,), jnp.float32),
               pltpu.VMEM((16,), jnp.float32),
               pltpu.VMEM((NW * 16,), jnp.float32),
               pltpu.VMEM_SHARED((3 * NW * 16,), jnp.float32)]),
    )
    pack, olab, oval = f(x1, y1, x2, y2, sp, lp)
    packed = pack[:150].reshape(30, 5)
    return packed, olab[:30], oval[:30].astype(bool)
